# Initial kernel scaffold; baseline (speedup 1.0000x reference)
import jax
import jax.numpy as jnp
from jax.experimental import pallas as pl


def kernel(x, edge_index, batch, W1, att_src1, att_dst1, b1, W2, att_src2, att_dst2, b2):
    raise NotImplementedError("write your pallas kernel here")


# speedup vs baseline: 8.0805x; 8.0805x over previous
import functools

import jax
import jax.numpy as jnp
from jax import lax
from jax.experimental import pallas as pl
from jax.experimental.pallas import tpu as pltpu
from jax.experimental.pallas import tpu_sc as plsc

NN = 10000
NP = 10240
EE = 160000
ETOT = EE + NN
IND = 256
HID = 128
NH = 8
OUTD = 128
NG = 64
DW = 16
PADNODE = 10200

NC = 2
NS = 16
L = 16
CH = 128
EPAD = 172032
HPC = NH // NC
EPT1 = EPAD // NS
NCH1 = EPT1 // CH
EPT2 = EPAD // (NC * NS)
NCH2 = EPT2 // CH
RPT = NP // NS

RB1 = 512
RB2 = 512
RB3 = 512
NB1 = NP // RB1
NB2 = NP // RB2
NB3 = NP // RB3

_mesh = plsc.VectorSubcoreMesh(core_axis_name="c", subcore_axis_name="s")
_sc_params = pltpu.CompilerParams(needs_layout_passes=False)


def _tc1_body(x_ref, w_ref, asv_ref, adv_ref, tab_ref, as_ref, ad_ref):
    h = jnp.dot(x_ref[...], w_ref[...], preferred_element_type=jnp.float32)
    hr = h.reshape(RB1, NH, HID)
    a_s = jnp.sum(hr * asv_ref[...][None], axis=-1)
    a_d = jnp.sum(hr * adv_ref[...][None], axis=-1)
    tab_ref[...] = hr.reshape(RB1 * NH, HID)
    as_ref[...] = a_s.T
    ad_ref[...] = a_d.T


def _tc1(x_p, W1, asv, adv):
    return pl.pallas_call(
        _tc1_body,
        grid=(NB1,),
        in_specs=[
            pl.BlockSpec((RB1, IND), lambda i: (i, 0)),
            pl.BlockSpec((IND, NH * HID), lambda i: (0, 0)),
            pl.BlockSpec((NH, HID), lambda i: (0, 0)),
            pl.BlockSpec((NH, HID), lambda i: (0, 0)),
        ],
        out_specs=[
            pl.BlockSpec((RB1 * NH, HID), lambda i: (i, 0)),
            pl.BlockSpec((NH, RB1), lambda i: (0, i)),
            pl.BlockSpec((NH, RB1), lambda i: (0, i)),
        ],
        out_shape=[
            jax.ShapeDtypeStruct((NP * NH, HID), jnp.float32),
            jax.ShapeDtypeStruct((NH, NP), jnp.float32),
            jax.ShapeDtypeStruct((NH, NP), jnp.float32),
        ],
    )(x_p, W1, asv, adv)


def _attn_w16(asv, adv, s16, d16):
    e16 = (plsc.load_gather(asv, [s16 >> 7, s16 & 127])
           + plsc.load_gather(adv, [d16 >> 7, d16 & 127]))
    e16 = jnp.where(e16 >= 0.0, e16, e16 * jnp.float32(0.2))
    return jnp.exp(e16)


def _fill_wrows(wrowv, iota16, j, w16):
    row16 = iota16 + (j * L)
    for p in range(DW):
        plsc.store_scatter(wrowv, [row16, jnp.full((L,), p, jnp.int32)], w16)


@functools.partial(
    pl.kernel,
    mesh=_mesh,
    compiler_params=_sc_params,
    out_type=[
        jax.ShapeDtypeStruct((NH * EPAD,), jnp.float32),
        jax.ShapeDtypeStruct((NH * NP, DW), jnp.float32),
    ],
    scratch_types=[
        pltpu.VMEM((CH,), jnp.int32),
        pltpu.VMEM((CH,), jnp.int32),
        pltpu.VMEM((CH,), jnp.float32),
        pltpu.VMEM((CH, DW), jnp.float32),
        pltpu.VMEM((NP // CH, CH), jnp.float32),
        pltpu.VMEM((NP // CH, CH), jnp.float32),
        pltpu.VMEM_SHARED((NP, DW), jnp.float32),
    ],
)
def _scw1(asrc, adst, src, dst, w_out, den_out,
          srcv, dstv, wv, wrowv, asv, adv, aggd):
    c = lax.axis_index("c")
    s = lax.axis_index("s")
    z16 = jnp.zeros((L,), jnp.float32)
    iota16 = lax.broadcasted_iota(jnp.int32, (L,), 0)

    for hi in range(HPC):
        k = c * HPC + hi

        def _zi(i, carry):
            wrowv[i, :] = z16
            return carry

        lax.fori_loop(0, CH, _zi, 0)
        for z in range(RPT // CH):
            pltpu.sync_copy(wrowv, aggd.at[pl.ds(s * RPT + z * CH, CH)])
        plsc.subcore_barrier()
        pltpu.sync_copy(asrc.at[k], asv)
        pltpu.sync_copy(adst.at[k], adv)

        def _chunk(g, carry):
            off = s * EPT1 + g * CH
            pltpu.sync_copy(src.at[pl.ds(off, CH)], srcv)
            pltpu.sync_copy(dst.at[pl.ds(off, CH)], dstv)
            for j in range(CH // L):
                s16 = srcv[pl.ds(j * L, L)]
                d16 = dstv[pl.ds(j * L, L)]
                w16 = _attn_w16(asv, adv, s16, d16)
                wv[pl.ds(j * L, L)] = w16
                _fill_wrows(wrowv, iota16, j, w16)
            pltpu.sync_copy(wrowv, aggd.at[dstv], add=True)
            pltpu.sync_copy(wv, w_out.at[pl.ds(k * EPAD + off, CH)])
            return carry

        lax.fori_loop(0, NCH1, _chunk, 0)
        plsc.subcore_barrier()
        pltpu.sync_copy(aggd.at[pl.ds(s * RPT, RPT)],
                        den_out.at[pl.ds(k * NP + s * RPT, RPT)])


@functools.partial(
    pl.kernel,
    mesh=_mesh,
    compiler_params=_sc_params,
    out_type=jax.ShapeDtypeStruct((NH * NP, HID), jnp.float32),
    scratch_types=[
        pltpu.VMEM((CH,), jnp.int32),
        pltpu.VMEM((CH,), jnp.int32),
        pltpu.VMEM((CH,), jnp.float32),
        pltpu.VMEM((CH,), jnp.int32),
        pltpu.VMEM((CH, HID), jnp.float32),
        pltpu.VMEM((CH, DW), jnp.float32),
        pltpu.VMEM_SHARED((NP, HID), jnp.float32),
    ],
)
def _sc1(tab, src, dst, w_in, val_out,
         srcv, dstv, wv, gidxv, rowsv, wrowv, aggv):
    c = lax.axis_index("c")
    s = lax.axis_index("s")
    z16 = jnp.zeros((L,), jnp.float32)
    iota16 = lax.broadcasted_iota(jnp.int32, (L,), 0)

    for hi in range(HPC):
        k = c * HPC + hi

        def _zi(i, carry):
            for q in range(HID // L):
                rowsv[i, pl.ds(q * L, L)] = z16
            return carry

        lax.fori_loop(0, CH, _zi, 0)
        for z in range(RPT // CH):
            pltpu.sync_copy(rowsv, aggv.at[pl.ds(s * RPT + z * CH, CH)])
        plsc.subcore_barrier()

        def _chunk(g, carry):
            off = s * EPT1 + g * CH
            pltpu.sync_copy(src.at[pl.ds(off, CH)], srcv)
            pltpu.sync_copy(dst.at[pl.ds(off, CH)], dstv)
            pltpu.sync_copy(w_in.at[pl.ds(k * EPAD + off, CH)], wv)
            for j in range(CH // L):
                s16 = srcv[pl.ds(j * L, L)]
                gidxv[pl.ds(j * L, L)] = s16 * NH + k
                _fill_wrows(wrowv, iota16, j, wv[pl.ds(j * L, L)])
            pltpu.sync_copy(tab.at[gidxv], rowsv)

            def _edge(j, cc):
                wvec = wrowv[j, :]
                for q in range(HID // L):
                    rowsv[j, pl.ds(q * L, L)] = rowsv[j, pl.ds(q * L, L)] * wvec
                return cc

            lax.fori_loop(0, CH, _edge, 0)
            pltpu.sync_copy(rowsv, aggv.at[dstv], add=True)
            return carry

        lax.fori_loop(0, NCH1, _chunk, 0)
        plsc.subcore_barrier()
        pltpu.sync_copy(aggv.at[pl.ds(s * RPT, RPT)],
                        val_out.at[pl.ds(k * NP + s * RPT, RPT)])


def _tc2_body(val_ref, den_ref, w2_ref, as2_ref, ad2_ref, b1_ref,
              tab2_ref, a2_ref):
    v = val_ref[...]
    d = den_ref[...][:, :, 0]
    h1 = v / (d[:, :, None] + 1e-16) + b1_ref[...][:, None, :]
    h1 = jnp.maximum(h1, 0.0)
    h2 = jnp.zeros((RB2, OUTD), jnp.float32)
    for k in range(NH):
        h2 = h2 + jnp.dot(h1[k], w2_ref[k], preferred_element_type=jnp.float32)
    a_s = jnp.sum(h2 * as2_ref[...], axis=-1)
    a_d = jnp.sum(h2 * ad2_ref[...], axis=-1)
    tab2_ref[...] = h2
    a2_ref[...] = jnp.concatenate(
        [a_s[None, :], a_d[None, :], jnp.zeros((6, RB2), jnp.float32)], axis=0)


def _tc2(val1, den1, W2r, as2, ad2, b1r):
    return pl.pallas_call(
        _tc2_body,
        grid=(NB2,),
        in_specs=[
            pl.BlockSpec((NH, RB2, HID), lambda i: (0, i, 0)),
            pl.BlockSpec((NH, RB2, DW), lambda i: (0, i, 0)),
            pl.BlockSpec((NH, HID, OUTD), lambda i: (0, 0, 0)),
            pl.BlockSpec((1, OUTD), lambda i: (0, 0)),
            pl.BlockSpec((1, OUTD), lambda i: (0, 0)),
            pl.BlockSpec((NH, HID), lambda i: (0, 0)),
        ],
        out_specs=[
            pl.BlockSpec((RB2, OUTD), lambda i: (i, 0)),
            pl.BlockSpec((NH, RB2), lambda i: (0, i)),
        ],
        out_shape=[
            jax.ShapeDtypeStruct((NP, OUTD), jnp.float32),
            jax.ShapeDtypeStruct((NH, NP), jnp.float32),
        ],
    )(val1, den1, W2r, as2, ad2, b1r)


@functools.partial(
    pl.kernel,
    mesh=_mesh,
    compiler_params=_sc_params,
    out_type=[
        jax.ShapeDtypeStruct((EPAD,), jnp.float32),
        jax.ShapeDtypeStruct((NC * NP, DW), jnp.float32),
    ],
    scratch_types=[
        pltpu.VMEM((CH,), jnp.int32),
        pltpu.VMEM((CH,), jnp.int32),
        pltpu.VMEM((CH,), jnp.float32),
        pltpu.VMEM((CH, DW), jnp.float32),
        pltpu.VMEM((NP // CH, CH), jnp.float32),
        pltpu.VMEM((NP // CH, CH), jnp.float32),
        pltpu.VMEM_SHARED((NP, DW), jnp.float32),
    ],
)
def _scw2(a2f, src, dst, w_out, den_out,
          srcv, dstv, wv, wrowv, asv, adv, aggd):
    c = lax.axis_index("c")
    s = lax.axis_index("s")
    z16 = jnp.zeros((L,), jnp.float32)
    iota16 = lax.broadcasted_iota(jnp.int32, (L,), 0)

    def _zi(i, carry):
        wrowv[i, :] = z16
        return carry

    lax.fori_loop(0, CH, _zi, 0)
    for z in range(RPT // CH):
        pltpu.sync_copy(wrowv, aggd.at[pl.ds(s * RPT + z * CH, CH)])
    plsc.subcore_barrier()
    pltpu.sync_copy(a2f.at[0], asv)
    pltpu.sync_copy(a2f.at[1], adv)

    ebase = (c * NS + s) * EPT2

    def _chunk(g, carry):
        off = ebase + g * CH
        pltpu.sync_copy(src.at[pl.ds(off, CH)], srcv)
        pltpu.sync_copy(dst.at[pl.ds(off, CH)], dstv)
        for j in range(CH // L):
            s16 = srcv[pl.ds(j * L, L)]
            d16 = dstv[pl.ds(j * L, L)]
            w16 = _attn_w16(asv, adv, s16, d16)
            wv[pl.ds(j * L, L)] = w16
            _fill_wrows(wrowv, iota16, j, w16)
        pltpu.sync_copy(wrowv, aggd.at[dstv], add=True)
        pltpu.sync_copy(wv, w_out.at[pl.ds(off, CH)])
        return carry

    lax.fori_loop(0, NCH2, _chunk, 0)
    plsc.subcore_barrier()
    pltpu.sync_copy(aggd.at[pl.ds(s * RPT, RPT)],
                    den_out.at[pl.ds(c * NP + s * RPT, RPT)])


@functools.partial(
    pl.kernel,
    mesh=_mesh,
    compiler_params=_sc_params,
    out_type=jax.ShapeDtypeStruct((NC * NP, HID), jnp.float32),
    scratch_types=[
        pltpu.VMEM((CH,), jnp.int32),
        pltpu.VMEM((CH,), jnp.int32),
        pltpu.VMEM((CH,), jnp.float32),
        pltpu.VMEM((CH, HID), jnp.float32),
        pltpu.VMEM((CH, DW), jnp.float32),
        pltpu.VMEM_SHARED((NP, HID), jnp.float32),
    ],
)
def _sc2(tab, src, dst, w_in, val_out,
         srcv, dstv, wv, rowsv, wrowv, aggv):
    c = lax.axis_index("c")
    s = lax.axis_index("s")
    z16 = jnp.zeros((L,), jnp.float32)
    iota16 = lax.broadcasted_iota(jnp.int32, (L,), 0)

    def _zi(i, carry):
        for q in range(HID // L):
            rowsv[i, pl.ds(q * L, L)] = z16
        return carry

    lax.fori_loop(0, CH, _zi, 0)
    for z in range(RPT // CH):
        pltpu.sync_copy(rowsv, aggv.at[pl.ds(s * RPT + z * CH, CH)])
    plsc.subcore_barrier()

    ebase = (c * NS + s) * EPT2

    def _chunk(g, carry):
        off = ebase + g * CH
        pltpu.sync_copy(src.at[pl.ds(off, CH)], srcv)
        pltpu.sync_copy(dst.at[pl.ds(off, CH)], dstv)
        pltpu.sync_copy(w_in.at[pl.ds(off, CH)], wv)
        for j in range(CH // L):
            _fill_wrows(wrowv, iota16, j, wv[pl.ds(j * L, L)])
        pltpu.sync_copy(tab.at[srcv], rowsv)

        def _edge(j, cc):
            wvec = wrowv[j, :]
            for q in range(HID // L):
                rowsv[j, pl.ds(q * L, L)] = rowsv[j, pl.ds(q * L, L)] * wvec
            return cc

        lax.fori_loop(0, CH, _edge, 0)
        pltpu.sync_copy(rowsv, aggv.at[dstv], add=True)
        return carry

    lax.fori_loop(0, NCH2, _chunk, 0)
    plsc.subcore_barrier()
    pltpu.sync_copy(aggv.at[pl.ds(s * RPT, RPT)],
                    val_out.at[pl.ds(c * NP + s * RPT, RPT)])


def _tc3_body(val_ref, den_ref, b2_ref, batch_ref, out_ref, acc, cnt):
    i = pl.program_id(0)

    @pl.when(i == 0)
    def _():
        acc[...] = jnp.zeros((NG, OUTD), jnp.float32)
        cnt[...] = jnp.zeros((NG, OUTD), jnp.float32)

    v = val_ref[...]
    d = den_ref[...]
    denom = d[0, :, 0:1] + d[1, :, 0:1] + 1e-16
    out2 = (v[0] + v[1]) / denom + b2_ref[...]
    b = batch_ref[...][0, 0, :]
    oh = (b[None, :] == lax.broadcasted_iota(jnp.int32, (NG, RB3), 0))
    oh = oh.astype(jnp.float32)
    acc[...] += jnp.dot(oh, out2, preferred_element_type=jnp.float32)
    cnt[...] += jnp.broadcast_to(jnp.sum(oh, axis=1, keepdims=True), (NG, OUTD))

    @pl.when(i == NB3 - 1)
    def _():
        out_ref[...] = acc[...] / jnp.maximum(cnt[...], 1.0)


def _tc3(val2, den2, b2r, batch3d):
    return pl.pallas_call(
        _tc3_body,
        grid=(NB3,),
        in_specs=[
            pl.BlockSpec((NC, RB3, HID), lambda i: (0, i, 0)),
            pl.BlockSpec((NC, RB3, DW), lambda i: (0, i, 0)),
            pl.BlockSpec((1, OUTD), lambda i: (0, 0)),
            pl.BlockSpec((1, 1, RB3), lambda i: (i, 0, 0)),
        ],
        out_specs=pl.BlockSpec((NG, OUTD), lambda i: (0, 0)),
        out_shape=jax.ShapeDtypeStruct((NG, OUTD), jnp.float32),
        scratch_shapes=[
            pltpu.VMEM((NG, OUTD), jnp.float32),
            pltpu.VMEM((NG, OUTD), jnp.float32),
        ],
    )(val2, den2, b2r, batch3d)


def kernel(x, edge_index, batch, W1, att_src1, att_dst1, b1,
           W2, att_src2, att_dst2, b2):
    loop = jnp.arange(NN, dtype=jnp.int32)
    pad = jnp.full((EPAD - ETOT,), PADNODE, jnp.int32)
    src = jnp.concatenate([edge_index[0].astype(jnp.int32), loop, pad])
    dst = jnp.concatenate([edge_index[1].astype(jnp.int32), loop, pad])
    x_p = jnp.pad(x, ((0, NP - NN), (0, 0)))
    batch3d = jnp.concatenate(
        [batch.astype(jnp.int32), jnp.full((NP - NN,), NG, jnp.int32)]
    ).reshape(NB3, 1, RB3)

    tab1, as1, ad1 = _tc1(x_p, W1, att_src1.reshape(NH, HID),
                          att_dst1.reshape(NH, HID))
    as1r = as1.reshape(NH, NP // CH, CH)
    ad1r = ad1.reshape(NH, NP // CH, CH)
    w1e, den1 = _scw1(as1r, ad1r, src, dst)
    val1 = _sc1(tab1, src, dst, w1e)
    tab2, a2 = _tc2(val1.reshape(NH, NP, HID), den1.reshape(NH, NP, DW),
                    W2.reshape(NH, HID, OUTD), att_src2.reshape(1, OUTD),
                    att_dst2.reshape(1, OUTD), b1.reshape(NH, HID))
    a2r = a2.reshape(NH, NP // CH, CH)
    w2e, den2 = _scw2(a2r, src, dst)
    val2 = _sc2(tab2, src, dst, w2e)
    out = _tc3(val2.reshape(NC, NP, HID), den2.reshape(NC, NP, DW),
               b2.reshape(1, OUTD), batch3d)
    return out
